# baseline (device time: 16753 ns/iter reference)
import jax
import jax.numpy as jnp
from jax import lax
from jax.experimental import pallas as pl
from jax.experimental.pallas import tpu as pltpu

N_DEV = 4
NCHUNK = 4


def kernel(x):
    m, n = x.shape
    q = m // 4
    qc = q // NCHUNK

    def body(x_ref, out_ref, xk_ref, comm_ref, w_ref, o_ref,
             load_sems, store_sems, send_sems, recv_sems):
        p = lax.axis_index("i")
        py = jnp.bitwise_xor(p, 1)
        px = jnp.bitwise_xor(p, 3)

        gray = jnp.bitwise_and(jnp.bitwise_xor(p, p // 2), 1)
        bit1 = jnp.bitwise_and(p // 2, 1)

        plans = [
            (py, px, 0, gray),
            (px, py, 2 * q, bit1),
        ]

        def branch_on(keep, fn):
            for kv in (0, 1):
                pl.when(keep == kv)(lambda kv=kv: fn(kv))

        loads = {}
        rdmas = {}
        stores = {}

        for b, (_p1, _p2, base, keep) in enumerate(plans):
            for c in range(NCHUNK):

                def start_load(kv, b=b, base=base, c=c):
                    row = base + kv * q + c * qc
                    pltpu.make_async_copy(
                        x_ref.at[pl.ds(row, qc)],
                        xk_ref.at[b, c],
                        load_sems.at[b, c],
                    ).start()

                branch_on(keep, start_load)
                loads[b, c] = pltpu.make_async_copy(
                    x_ref.at[pl.ds(base, qc)],
                    xk_ref.at[b, c],
                    load_sems.at[b, c],
                )

        barrier_sem = pltpu.get_barrier_semaphore()
        for nbr in (py, px):
            pl.semaphore_signal(
                barrier_sem, inc=1,
                device_id=(nbr,), device_id_type=pl.DeviceIdType.MESH,
            )
        pl.semaphore_wait(barrier_sem, 2)

        for b, (p1, _p2, base, keep) in enumerate(plans):
            for c in range(NCHUNK):

                def start_s1(kv, b=b, p1=p1, base=base, c=c):
                    row = base + (1 - kv) * q + c * qc
                    pltpu.make_async_remote_copy(
                        src_ref=x_ref.at[pl.ds(row, qc)],
                        dst_ref=comm_ref.at[b, 0, c],
                        send_sem=send_sems.at[b, 0, c],
                        recv_sem=recv_sems.at[b, 0, c],
                        device_id=(p1,),
                        device_id_type=pl.DeviceIdType.MESH,
                    ).start()

                branch_on(keep, start_s1)
                rdmas[b, 0, c] = pltpu.make_async_remote_copy(
                    src_ref=x_ref.at[pl.ds(base, qc)],
                    dst_ref=comm_ref.at[b, 0, c],
                    send_sem=send_sems.at[b, 0, c],
                    recv_sem=recv_sems.at[b, 0, c],
                    device_id=(p1,),
                    device_id_type=pl.DeviceIdType.MESH,
                )

        for c in range(NCHUNK):
            for b, (p1, p2, base, keep) in enumerate(plans):
                loads[b, c].wait()
                rdmas[b, 0, c].wait_recv()
                w_ref[b, c, :, :] = xk_ref[b, c, :, :] + comm_ref[b, 0, c, :, :]
                r = pltpu.make_async_remote_copy(
                    src_ref=w_ref.at[b, c],
                    dst_ref=comm_ref.at[b, 1, c],
                    send_sem=send_sems.at[b, 1, c],
                    recv_sem=recv_sems.at[b, 1, c],
                    device_id=(p2,),
                    device_id_type=pl.DeviceIdType.MESH,
                )
                r.start()
                rdmas[b, 1, c] = r

        for c in range(NCHUNK):
            for b, (p1, _p2, base, keep) in enumerate(plans):
                rdmas[b, 1, c].wait_recv()
                o_ref[b, c, :, :] = w_ref[b, c, :, :] + comm_ref[b, 1, c, :, :]

                def start_s3(kv, b=b, p1=p1, base=base, c=c):
                    row = base + kv * q + c * qc
                    pltpu.make_async_remote_copy(
                        src_ref=o_ref.at[b, c],
                        dst_ref=out_ref.at[pl.ds(row, qc)],
                        send_sem=send_sems.at[b, 2, c],
                        recv_sem=recv_sems.at[b, 2, c],
                        device_id=(p1,),
                        device_id_type=pl.DeviceIdType.MESH,
                    ).start()
                    pltpu.make_async_copy(
                        o_ref.at[b, c],
                        out_ref.at[pl.ds(row, qc)],
                        store_sems.at[b, c],
                    ).start()

                branch_on(keep, start_s3)
                rdmas[b, 2, c] = pltpu.make_async_remote_copy(
                    src_ref=o_ref.at[b, c],
                    dst_ref=out_ref.at[pl.ds(base, qc)],
                    send_sem=send_sems.at[b, 2, c],
                    recv_sem=recv_sems.at[b, 2, c],
                    device_id=(p1,),
                    device_id_type=pl.DeviceIdType.MESH,
                )
                stores[b, c] = pltpu.make_async_copy(
                    o_ref.at[b, c],
                    out_ref.at[pl.ds(base, qc)],
                    store_sems.at[b, c],
                )

        for c in range(NCHUNK):
            for b in range(2):
                rdmas[b, 2, c].wait_recv()
                stores[b, c].wait()
        for r in rdmas.values():
            r.wait_send()

    return pl.pallas_call(
        body,
        out_shape=jax.ShapeDtypeStruct((m, n), x.dtype),
        in_specs=[pl.BlockSpec(memory_space=pl.ANY)],
        out_specs=pl.BlockSpec(memory_space=pl.ANY),
        scratch_shapes=[
            pltpu.VMEM((2, NCHUNK, qc, n), x.dtype),
            pltpu.VMEM((2, 2, NCHUNK, qc, n), x.dtype),
            pltpu.VMEM((2, NCHUNK, qc, n), x.dtype),
            pltpu.VMEM((2, NCHUNK, qc, n), x.dtype),
            pltpu.SemaphoreType.DMA((2, NCHUNK)),
            pltpu.SemaphoreType.DMA((2, NCHUNK)),
            pltpu.SemaphoreType.DMA((2, 3, NCHUNK)),
            pltpu.SemaphoreType.DMA((2, 3, NCHUNK)),
        ],
        compiler_params=pltpu.CompilerParams(collective_id=0),
    )(x)


# device time: 15942 ns/iter; 1.0509x vs baseline; 1.0509x over previous
import jax
import jax.numpy as jnp
from jax import lax
from jax.experimental import pallas as pl
from jax.experimental.pallas import tpu as pltpu

N_DEV = 4
NCHUNK = 4


def kernel(x):
    m, n = x.shape
    q = m // 4
    qc = q // NCHUNK

    def body(x_ref, out_ref, comm_ref, w_ref, send_sems, recv_sems):
        p = lax.axis_index("i")
        py = jnp.bitwise_xor(p, 1)
        px = jnp.bitwise_xor(p, 3)

        barrier_sem = pltpu.get_barrier_semaphore()
        for nbr in (py, px):
            pl.semaphore_signal(
                barrier_sem, inc=1,
                device_id=(nbr,), device_id_type=pl.DeviceIdType.MESH,
            )
        pl.semaphore_wait(barrier_sem, 2)

        gray = jnp.bitwise_and(jnp.bitwise_xor(p, p // 2), 1)
        bit1 = jnp.bitwise_and(p // 2, 1)

        plans = [
            (py, px, 0, gray),
            (px, py, 2 * q, bit1),
        ]

        def branch_on(keep, fn):
            for kv in (0, 1):
                pl.when(keep == kv)(lambda kv=kv: fn(kv))

        rdmas = {}

        for b, (p1, _p2, base, keep) in enumerate(plans):
            for c in range(NCHUNK):

                def start_s1(kv, b=b, p1=p1, base=base, c=c):
                    src_row = base + (1 - kv) * q + c * qc
                    pltpu.make_async_remote_copy(
                        src_ref=x_ref.at[pl.ds(src_row, qc)],
                        dst_ref=comm_ref.at[b, 0, c],
                        send_sem=send_sems.at[b, 0, c],
                        recv_sem=recv_sems.at[b, 0, c],
                        device_id=(p1,),
                        device_id_type=pl.DeviceIdType.MESH,
                    ).start()

                branch_on(keep, start_s1)
                rdmas[b, 0, c] = pltpu.make_async_remote_copy(
                    src_ref=x_ref.at[pl.ds(base, qc)],
                    dst_ref=comm_ref.at[b, 0, c],
                    send_sem=send_sems.at[b, 0, c],
                    recv_sem=recv_sems.at[b, 0, c],
                    device_id=(p1,),
                    device_id_type=pl.DeviceIdType.MESH,
                )

        for c in range(NCHUNK):
            for b, (p1, p2, base, keep) in enumerate(plans):
                rdmas[b, 0, c].wait_recv()

                def reduce_s1(kv, b=b, base=base, c=c):
                    row = base + kv * q + c * qc
                    w_ref[b, c, :, :] = (
                        x_ref[row:row + qc, :] + comm_ref[b, 0, c, :, :]
                    )

                branch_on(keep, reduce_s1)
                r = pltpu.make_async_remote_copy(
                    src_ref=w_ref.at[b, c],
                    dst_ref=comm_ref.at[b, 1, c],
                    send_sem=send_sems.at[b, 1, c],
                    recv_sem=recv_sems.at[b, 1, c],
                    device_id=(p2,),
                    device_id_type=pl.DeviceIdType.MESH,
                )
                r.start()
                rdmas[b, 1, c] = r

        for c in range(NCHUNK):
            for b, (p1, _p2, base, keep) in enumerate(plans):
                rdmas[b, 1, c].wait_recv()

                def reduce_and_s3(kv, b=b, p1=p1, base=base, c=c):
                    row = base + kv * q + c * qc
                    out_ref[row:row + qc, :] = (
                        w_ref[b, c, :, :] + comm_ref[b, 1, c, :, :]
                    )
                    pltpu.make_async_remote_copy(
                        src_ref=out_ref.at[pl.ds(row, qc)],
                        dst_ref=out_ref.at[pl.ds(row, qc)],
                        send_sem=send_sems.at[b, 2, c],
                        recv_sem=recv_sems.at[b, 2, c],
                        device_id=(p1,),
                        device_id_type=pl.DeviceIdType.MESH,
                    ).start()

                branch_on(keep, reduce_and_s3)
                rdmas[b, 2, c] = pltpu.make_async_remote_copy(
                    src_ref=out_ref.at[pl.ds(base, qc)],
                    dst_ref=out_ref.at[pl.ds(base, qc)],
                    send_sem=send_sems.at[b, 2, c],
                    recv_sem=recv_sems.at[b, 2, c],
                    device_id=(p1,),
                    device_id_type=pl.DeviceIdType.MESH,
                )

        for c in range(NCHUNK):
            for b in range(2):
                rdmas[b, 2, c].wait_recv()
        for r in rdmas.values():
            r.wait_send()

    return pl.pallas_call(
        body,
        out_shape=jax.ShapeDtypeStruct((m, n), x.dtype),
        in_specs=[pl.BlockSpec(memory_space=pltpu.VMEM)],
        out_specs=pl.BlockSpec(memory_space=pltpu.VMEM),
        scratch_shapes=[
            pltpu.VMEM((2, 2, NCHUNK, qc, n), x.dtype),
            pltpu.VMEM((2, NCHUNK, qc, n), x.dtype),
            pltpu.SemaphoreType.DMA((2, 3, NCHUNK)),
            pltpu.SemaphoreType.DMA((2, 3, NCHUNK)),
        ],
        compiler_params=pltpu.CompilerParams(collective_id=0),
    )(x)
